# Initial kernel scaffold; baseline (speedup 1.0000x reference)
import jax
import jax.numpy as jnp
from jax.experimental import pallas as pl


def kernel(categorical_item, categorical_category, numerical_price, text_tags, W_item, W_cat, Wn, bn, Wt, Wp, bp, gamma, beta, fw):
    raise NotImplementedError("write your pallas kernel here")


# speedup vs baseline: 4.6423x; 4.6423x over previous
import functools

import jax
import jax.numpy as jnp
from jax import lax
from jax.experimental import pallas as pl
from jax.experimental.pallas import tpu as pltpu
from jax.experimental.pallas import tpu_sc as plsc

B = 16384
L = 50
D = 128
D4 = D // 4
D8 = D // 8

NC = 2
NS = 16
NW = NC * NS
RPW = B // NW
CHUNK = 128
RB = 8
NBLK = RPW // RB

_mesh = plsc.VectorSubcoreMesh(
    core_axis_name="c", subcore_axis_name="s", num_cores=NC, num_subcores=NS)


@functools.partial(
    pl.kernel,
    out_type=(
        jax.ShapeDtypeStruct((B, D4), jnp.float32),
        jax.ShapeDtypeStruct((B, D4), jnp.float32),
        jax.ShapeDtypeStruct((B, D4), jnp.float32),
    ),
    mesh=_mesh,
    compiler_params=pltpu.CompilerParams(use_tc_tiling_on_sc=False),
    scratch_types=[
        pltpu.VMEM((CHUNK,), jnp.int32),
        pltpu.VMEM((CHUNK, D4), jnp.float32),
        pltpu.VMEM((RB, L), jnp.int32),
        pltpu.VMEM((RB, L, D4), jnp.float32),
        pltpu.VMEM((RB, D4), jnp.float32),
        pltpu.SemaphoreType.DMA,
    ],
)
def _sc_encode(item_idx_hbm, cat_idx_hbm, tags_hbm, w_item_hbm, w_cat_hbm,
               wt_hbm, e_item_hbm, e_cat_hbm, e_tsum_hbm,
               idx_v, rows_v, tag_v, trows_v, bsum_v, sem):
    wid = lax.axis_index("s") * NC + lax.axis_index("c")
    base = wid * RPW

    for src_idx, table, out in (
        (item_idx_hbm, w_item_hbm, e_item_hbm),
        (cat_idx_hbm, w_cat_hbm, e_cat_hbm),
    ):
        def cat_body(c, carry, src_idx=src_idx, table=table, out=out):
            r0 = pl.multiple_of(base + c * CHUNK, CHUNK)
            pltpu.sync_copy(src_idx.at[pl.ds(r0, CHUNK)], idx_v)
            pltpu.async_copy(table.at[idx_v], rows_v, sem).wait()
            pltpu.sync_copy(rows_v, out.at[pl.ds(r0, CHUNK)])
            return carry
        lax.fori_loop(0, RPW // CHUNK, cat_body, 0)

    def blk_body(b, carry):
        r0 = pl.multiple_of(base + b * RB, RB)
        pltpu.sync_copy(tags_hbm.at[pl.ds(r0, RB)], tag_v)
        cps = [
            pltpu.async_copy(wt_hbm.at[tag_v.at[i]], trows_v.at[i], sem)
            for i in range(RB)
        ]
        for cp in cps:
            cp.wait()

        def bag_body(i, c2):
            row = trows_v.at[i]
            for h in range(2):
                s = pl.ds(16 * h, 16)
                accs = [row[j, s] for j in range(4)]
                for j in range(4, L):
                    accs[j % 4] = accs[j % 4] + row[j, s]
                bsum_v[i, s] = (accs[0] + accs[1]) + (accs[2] + accs[3])
            return c2
        lax.fori_loop(0, RB, bag_body, 0)
        pltpu.sync_copy(bsum_v, e_tsum_hbm.at[pl.ds(r0, RB)])
        return carry
    lax.fori_loop(0, NBLK, blk_body, 0)


def _tc_body(ei, ec, et, pr, wpa, wpb, wpc, wpd, wn, bn, bp, g, bt, o):
    dot = functools.partial(
        lax.dot, precision=lax.Precision.HIGHEST,
        preferred_element_type=jnp.float32)
    num = jnp.maximum(pr[...] * wn[...] + bn[...], 0.0)
    x = (dot(ei[...], wpa[...]) + dot(ec[...], wpb[...])
         + dot(et[...], wpc[...]) + dot(num, wpd[...]) + bp[...])
    mu = jnp.mean(x, axis=-1, keepdims=True)
    xc = x - mu
    var = jnp.mean(xc * xc, axis=-1, keepdims=True)
    o[...] = xc * lax.rsqrt(var + 1e-5) * g[...] + bt[...]


BT = 2048


def _tc_project(ei, ec, et, price, wpa, wpb, wpc, wpd, wn, bn, bp, gamma, beta):
    row_spec = pl.BlockSpec((BT, D4), lambda i: (i, 0))
    full = lambda shape: pl.BlockSpec(shape, lambda i: (0, 0))
    return pl.pallas_call(
        _tc_body,
        grid=(B // BT,),
        in_specs=[
            row_spec, row_spec, row_spec,
            pl.BlockSpec((BT, 1), lambda i: (i, 0)),
            full((D4, D)), full((D4, D)), full((D4, D)), full((D8, D)),
            full((1, D8)), full((1, D8)),
            full((1, D)), full((1, D)), full((1, D)),
        ],
        out_specs=pl.BlockSpec((BT, D), lambda i: (i, 0)),
        out_shape=jax.ShapeDtypeStruct((B, D), jnp.float32),
    )(ei, ec, et, price, wpa, wpb, wpc, wpd, wn, bn, bp, gamma, beta)


def kernel(categorical_item, categorical_category, numerical_price, text_tags,
           W_item, W_cat, Wn, bn, Wt, Wp, bp, gamma, beta, fw):
    item_idx = categorical_item.astype(jnp.int32)
    cat_idx = categorical_category.astype(jnp.int32)
    tags = text_tags.astype(jnp.int32)
    e_item, e_cat, e_tsum = _sc_encode(item_idx, cat_idx, tags, W_item, W_cat, Wt)
    wpa = Wp[0:D4] * fw[0]
    wpb = Wp[D4:2 * D4] * fw[1]
    wpd = Wp[2 * D4:2 * D4 + D8] * fw[2]
    wpc = Wp[2 * D4 + D8:] * (fw[3] / L)
    price = numerical_price.astype(jnp.float32).reshape(B, 1)
    return _tc_project(
        e_item, e_cat, e_tsum, price, wpa, wpb, wpc, wpd,
        Wn, bn.reshape(1, D8), bp.reshape(1, D),
        gamma.reshape(1, D), beta.reshape(1, D))


# speedup vs baseline: 5.4157x; 1.1666x over previous
import functools

import jax
import jax.numpy as jnp
from jax import lax
from jax.experimental import pallas as pl
from jax.experimental.pallas import tpu as pltpu
from jax.experimental.pallas import tpu_sc as plsc

B = 16384
L = 50
D = 128
D4 = D // 4
D8 = D // 8

NC = 2
NS = 16
NW = NC * NS
RPW = B // NW
CHUNK = 128
RB = 8
NBLK = RPW // RB

_mesh = plsc.VectorSubcoreMesh(
    core_axis_name="c", subcore_axis_name="s", num_cores=NC, num_subcores=NS)


@functools.partial(
    pl.kernel,
    out_type=jax.ShapeDtypeStruct((B, D4), jnp.float32),
    mesh=_mesh,
    compiler_params=pltpu.CompilerParams(use_tc_tiling_on_sc=False),
    scratch_types=[
        pltpu.VMEM((RB, L), jnp.int32),
        pltpu.VMEM((RB, L, D4), jnp.float32),
        pltpu.VMEM((RB, D4), jnp.float32),
        pltpu.SemaphoreType.DMA,
    ],
)
def _sc_text(tags_hbm, wt_hbm, e_tsum_hbm, tag_v, trows_v, bsum_v, sem):
    wid = lax.axis_index("s") * NC + lax.axis_index("c")
    base = wid * RPW

    def blk_body(b, carry):
        r0 = pl.multiple_of(base + b * RB, RB)
        pltpu.sync_copy(tags_hbm.at[pl.ds(r0, RB)], tag_v)
        cps = [
            pltpu.async_copy(wt_hbm.at[tag_v.at[i]], trows_v.at[i], sem)
            for i in range(RB)
        ]
        for cp in cps:
            cp.wait()

        def bag_body(i, c2):
            row = trows_v.at[i]
            for h in range(2):
                s = pl.ds(16 * h, 16)
                accs = [row[j, s] for j in range(4)]
                for j in range(4, L):
                    accs[j % 4] = accs[j % 4] + row[j, s]
                bsum_v[i, s] = (accs[0] + accs[1]) + (accs[2] + accs[3])
            return c2
        lax.fori_loop(0, RB, bag_body, 0)
        pltpu.sync_copy(bsum_v, e_tsum_hbm.at[pl.ds(r0, RB)])
        return carry
    lax.fori_loop(0, NBLK, blk_body, 0)


@functools.partial(
    pl.kernel,
    out_type=(
        jax.ShapeDtypeStruct((B, D), jnp.float32),
        jax.ShapeDtypeStruct((B, D), jnp.float32),
    ),
    mesh=_mesh,
    scratch_types=[
        pltpu.VMEM((RPW,), jnp.int32),
        pltpu.VMEM((CHUNK, D), jnp.float32),
        pltpu.SemaphoreType.DMA,
    ],
)
def _sc_cats(item_q_hbm, cat_q_hbm, w_item4_hbm, w_cat4_hbm,
             item_rows_hbm, cat_rows_hbm, idx_v, rows_v, sem):
    wid = lax.axis_index("s") * NC + lax.axis_index("c")
    base = wid * RPW
    for src_idx, table, out in (
        (item_q_hbm, w_item4_hbm, item_rows_hbm),
        (cat_q_hbm, w_cat4_hbm, cat_rows_hbm),
    ):
        pltpu.sync_copy(src_idx.at[pl.ds(base, RPW)], idx_v)

        def cat_body(c, carry, table=table, out=out):
            r0 = pl.multiple_of(base + c * CHUNK, CHUNK)
            pltpu.async_copy(
                table.at[idx_v.at[pl.ds(c * CHUNK, CHUNK)]], rows_v, sem
            ).wait()
            pltpu.sync_copy(rows_v, out.at[pl.ds(r0, CHUNK)])
            return carry
        lax.fori_loop(0, RPW // CHUNK, cat_body, 0)


def _quarter_select(rows, q):
    out = jnp.where(q == 0, rows[:, 0:D4], 0.0)
    for k in range(1, 4):
        out = out + jnp.where(q == k, rows[:, k * D4:(k + 1) * D4], 0.0)
    return out


def _tc_body(irows, crows, et, pr, iq, cq, wpa, wpb, wpc, wpd, wn, bn, bp,
             g, bt, o):
    dot = functools.partial(
        lax.dot, precision=lax.Precision.HIGHEST,
        preferred_element_type=jnp.float32)
    ei = _quarter_select(irows[...], iq[...])
    ec = _quarter_select(crows[...], cq[...])
    num = jnp.maximum(pr[...] * wn[...] + bn[...], 0.0)
    x = (dot(ei, wpa[...]) + dot(ec, wpb[...])
         + dot(et[...], wpc[...]) + dot(num, wpd[...]) + bp[...])
    mu = jnp.mean(x, axis=-1, keepdims=True)
    xc = x - mu
    var = jnp.mean(xc * xc, axis=-1, keepdims=True)
    o[...] = xc * lax.rsqrt(var + 1e-5) * g[...] + bt[...]


BT = 2048


def _tc_project(irows, crows, et, price, iq, cq, wpa, wpb, wpc, wpd, wn, bn,
                bp, gamma, beta):
    full = lambda shape: pl.BlockSpec(shape, lambda i: (0, 0))
    return pl.pallas_call(
        _tc_body,
        grid=(B // BT,),
        in_specs=[
            pl.BlockSpec((BT, D), lambda i: (i, 0)),
            pl.BlockSpec((BT, D), lambda i: (i, 0)),
            pl.BlockSpec((BT, D4), lambda i: (i, 0)),
            pl.BlockSpec((BT, 1), lambda i: (i, 0)),
            pl.BlockSpec((BT, 1), lambda i: (i, 0)),
            pl.BlockSpec((BT, 1), lambda i: (i, 0)),
            full((D4, D)), full((D4, D)), full((D4, D)), full((D8, D)),
            full((1, D8)), full((1, D8)),
            full((1, D)), full((1, D)), full((1, D)),
        ],
        out_specs=pl.BlockSpec((BT, D), lambda i: (i, 0)),
        out_shape=jax.ShapeDtypeStruct((B, D), jnp.float32),
    )(irows, crows, et, price, iq, cq, wpa, wpb, wpc, wpd, wn, bn, bp,
      gamma, beta)


def kernel(categorical_item, categorical_category, numerical_price, text_tags,
           W_item, W_cat, Wn, bn, Wt, Wp, bp, gamma, beta, fw):
    item_idx = categorical_item.astype(jnp.int32)
    cat_idx = categorical_category.astype(jnp.int32)
    tags = text_tags.astype(jnp.int32)
    e_tsum = _sc_text(tags, Wt)
    w_item4 = W_item.reshape(-1, D)
    w_cat4 = W_cat.reshape(-1, D)
    irows, crows = _sc_cats(
        jnp.right_shift(item_idx, 2), jnp.right_shift(cat_idx, 2),
        w_item4, w_cat4)
    wpa = Wp[0:D4] * fw[0]
    wpb = Wp[D4:2 * D4] * fw[1]
    wpd = Wp[2 * D4:2 * D4 + D8] * fw[2]
    wpc = Wp[2 * D4 + D8:] * (fw[3] / L)
    price = numerical_price.astype(jnp.float32).reshape(B, 1)
    iq = jnp.bitwise_and(item_idx, 3).reshape(B, 1)
    cq = jnp.bitwise_and(cat_idx, 3).reshape(B, 1)
    return _tc_project(
        irows, crows, e_tsum, price, iq, cq, wpa, wpb, wpc, wpd,
        Wn, bn.reshape(1, D8), bp.reshape(1, D),
        gamma.reshape(1, D), beta.reshape(1, D))
